# baseline (device time: 82362 ns/iter reference)
import jax
import jax.numpy as jnp
from jax import lax
from jax.experimental import pallas as pl
from jax.experimental.pallas import tpu as pltpu

N_DEV = 16


def kernel(x, w_mat):
    m_tot, k_blk = x.shape
    k_tot, n_tot = w_mat.shape
    m_blk = m_tot // N_DEV
    assert k_tot == N_DEV * k_blk

    def body(x_ref, w_ref, out_ref, recv_buf, send_sems, recv_sems):
        t = pl.program_id(0)
        my = lax.axis_index("i")

        @pl.when(t == 0)
        def _issue_all():
            local = pltpu.make_async_copy(
                x_ref.at[pl.ds(my * m_blk, m_blk), :],
                recv_buf.at[my],
                recv_sems.at[my],
            )
            local.start()
            for o in range(1, N_DEV):
                d = lax.rem(my + o, N_DEV)
                rdma = pltpu.make_async_remote_copy(
                    src_ref=x_ref.at[pl.ds(d * m_blk, m_blk), :],
                    dst_ref=recv_buf.at[my],
                    send_sem=send_sems.at[o],
                    recv_sem=recv_sems.at[my],
                    device_id=(d,),
                    device_id_type=pl.DeviceIdType.MESH,
                )
                rdma.start()

        pltpu.make_async_remote_copy(
            src_ref=x_ref.at[pl.ds(0, m_blk), :],
            dst_ref=recv_buf.at[t],
            send_sem=send_sems.at[0],
            recv_sem=recv_sems.at[t],
            device_id=(my,),
            device_id_type=pl.DeviceIdType.MESH,
        ).wait_recv()

        contrib = jnp.dot(
            recv_buf[t], w_ref[...], preferred_element_type=jnp.float32
        )

        @pl.when(t == 0)
        def _():
            out_ref[...] = contrib

        @pl.when(jnp.logical_and(t > 0, t < N_DEV - 1))
        def _():
            out_ref[...] += contrib

        @pl.when(t == N_DEV - 1)
        def _():
            y = out_ref[...] + contrib
            out_ref[...] = y * jax.nn.sigmoid(y)
            for o in range(1, N_DEV):
                pltpu.make_async_remote_copy(
                    src_ref=x_ref.at[pl.ds(0, m_blk), :],
                    dst_ref=recv_buf.at[0],
                    send_sem=send_sems.at[o],
                    recv_sem=recv_sems.at[0],
                    device_id=(my,),
                    device_id_type=pl.DeviceIdType.MESH,
                ).wait_send()

    return pl.pallas_call(
        body,
        grid=(N_DEV,),
        in_specs=[
            pl.BlockSpec((m_tot, k_blk), lambda t: (0, 0)),
            pl.BlockSpec((k_blk, n_tot), lambda t: (t, 0)),
        ],
        out_specs=pl.BlockSpec((m_blk, n_tot), lambda t: (0, 0)),
        out_shape=jax.ShapeDtypeStruct((m_blk, n_tot), jnp.float32),
        scratch_shapes=[
            pltpu.VMEM((N_DEV, m_blk, k_blk), jnp.float32),
            pltpu.SemaphoreType.DMA((N_DEV,)),
            pltpu.SemaphoreType.DMA((N_DEV,)),
        ],
        compiler_params=pltpu.CompilerParams(
            dimension_semantics=("arbitrary",),
            vmem_limit_bytes=110 * 1024 * 1024,
        ),
    )(x, w_mat)
